# baseline (speedup 1.0000x reference)
import jax
import jax.numpy as jnp
import numpy as np
from jax.experimental import pallas as pl

_ATOMS = 51
_V_MAX = 10.0
_V_MIN = -10.0
_SKEW = 0.0


def _proj_matrix():
    atoms = _ATOMS
    delta = np.float32((_V_MAX - _V_MIN) / (atoms - 1))
    supports = np.linspace(_V_MIN, _V_MAX, atoms).astype(np.float32)
    tz = np.clip(np.float32(_SKEW) + supports, _V_MIN, _V_MAX).astype(np.float32)
    b = ((tz - np.float32(_V_MIN)) / delta).astype(np.float32)
    l = np.floor(b)
    u = np.ceil(b)
    l = np.where((u > 0) & (l == u), l - 1.0, l).astype(np.float32)
    u = np.where((l < atoms - 1) & (l == u), u + 1.0, u).astype(np.float32)
    w = np.zeros((atoms, atoms), dtype=np.float32)
    for j in range(atoms):
        w[j, int(l[j])] += np.float32(u[j] - b[j])
        w[j, int(u[j])] += np.float32(b[j] - l[j])
    return w


def _body(anchor_ref, feature_ref, w_ref, out_ref):
    i = pl.program_id(0)
    logf = jnp.log(feature_ref[...] + 1e-16)
    proj = jax.lax.dot_general(
        anchor_ref[...], w_ref[...],
        dimension_numbers=(((1,), (0,)), ((), ())),
        preferred_element_type=jnp.float32,
    )
    partial = jnp.sum(proj * logf, keepdims=True)

    @pl.when(i == 0)
    def _init():
        out_ref[...] = jnp.zeros_like(out_ref)

    out_ref[...] += partial


def kernel(anchor, feature):
    batch, atoms = anchor.shape
    w = jnp.asarray(_proj_matrix())
    bm = 1024
    grid = batch // bm
    total = pl.pallas_call(
        _body,
        grid=(grid,),
        in_specs=[
            pl.BlockSpec((bm, atoms), lambda i: (i, 0)),
            pl.BlockSpec((bm, atoms), lambda i: (i, 0)),
            pl.BlockSpec((atoms, atoms), lambda i: (0, 0)),
        ],
        out_specs=pl.BlockSpec((1, 1), lambda i: (0, 0)),
        out_shape=jax.ShapeDtypeStruct((1, 1), jnp.float32),
    )(anchor, feature, w)
    return (-total[0, 0] / batch).astype(jnp.float32)


# speedup vs baseline: 1.3376x; 1.3376x over previous
import jax
import jax.numpy as jnp
import numpy as np
from jax.experimental import pallas as pl

_ATOMS = 51
_V_MAX = 10.0
_V_MIN = -10.0
_SKEW = 0.0


def _proj_matrix():
    atoms = _ATOMS
    delta = np.float32((_V_MAX - _V_MIN) / (atoms - 1))
    supports = np.linspace(_V_MIN, _V_MAX, atoms).astype(np.float32)
    tz = np.clip(np.float32(_SKEW) + supports, _V_MIN, _V_MAX).astype(np.float32)
    b = ((tz - np.float32(_V_MIN)) / delta).astype(np.float32)
    l = np.floor(b)
    u = np.ceil(b)
    l = np.where((u > 0) & (l == u), l - 1.0, l).astype(np.float32)
    u = np.where((l < atoms - 1) & (l == u), u + 1.0, u).astype(np.float32)
    w = np.zeros((atoms, atoms), dtype=np.float32)
    for j in range(atoms):
        w[j, int(l[j])] += np.float32(u[j] - b[j])
        w[j, int(u[j])] += np.float32(b[j] - l[j])
    return w


def _body(anchor_ref, feature_ref, w_ref, out_ref):
    i = pl.program_id(0)
    logf = jnp.log(feature_ref[...] + 1e-16)
    proj = jax.lax.dot_general(
        anchor_ref[...], w_ref[...],
        dimension_numbers=(((1,), (0,)), ((), ())),
        preferred_element_type=jnp.float32,
    )
    partial = jnp.sum(proj * logf, keepdims=True)

    @pl.when(i == 0)
    def _init():
        out_ref[...] = jnp.zeros_like(out_ref)

    out_ref[...] += partial


def kernel(anchor, feature):
    batch, atoms = anchor.shape
    w = jnp.asarray(_proj_matrix())
    bm = 8192
    grid = batch // bm
    total = pl.pallas_call(
        _body,
        grid=(grid,),
        in_specs=[
            pl.BlockSpec((bm, atoms), lambda i: (i, 0)),
            pl.BlockSpec((bm, atoms), lambda i: (i, 0)),
            pl.BlockSpec((atoms, atoms), lambda i: (0, 0)),
        ],
        out_specs=pl.BlockSpec((1, 1), lambda i: (0, 0)),
        out_shape=jax.ShapeDtypeStruct((1, 1), jnp.float32),
    )(anchor, feature, w)
    return (-total[0, 0] / batch).astype(jnp.float32)
